# baseline (speedup 1.0000x reference)
import jax
import jax.numpy as jnp
from jax.experimental import pallas as pl
from jax.experimental.pallas import tpu as pltpu

N = 300
H, W = 128, 224
HB = 8
NB = 8
SELECT_THR = 0.1
NMS_THR = 0.6


def _tracker_kernel(ms_ref, x0_ref, x1_ref, out_ref, sig_scr, inter_scr,
                    p_scr, w_scr, iou_scr, obuf_scr, sem):
    g = pl.program_id(0)
    f32 = jnp.float32

    @pl.when(g < NB)
    def _phase_a():
        part = None
        for s, x_ref in ((0, x0_ref), (1, x1_ref)):
            x = x_ref[...]
            sig_scr[s * NB + g] = jax.nn.sigmoid(x).astype(jnp.bfloat16)
            b = (x > 0.0).astype(jnp.bfloat16).reshape(N, HB * W)
            d = jax.lax.dot_general(
                b, b, (((1,), (1,)), ((), ())), preferred_element_type=f32)
            part = d if part is None else part + d

        @pl.when(g == 0)
        def _():
            inter_scr[...] = part

        @pl.when(g > 0)
        def _():
            inter_scr[...] += part

    @pl.when(g == NB)
    def _phase_b():
        ms = ms_ref[...]
        inter = inter_scr[...]
        row_i = jax.lax.broadcasted_iota(jnp.int32, (N, N), 0)
        col_i = jax.lax.broadcasted_iota(jnp.int32, (N, N), 1)
        eye = (row_i == col_i).astype(f32)
        hi = jax.lax.Precision.HIGHEST

        def to_col(r):
            return jax.lax.dot_general(
                eye, r, (((1,), (1,)), ((), ())), preferred_element_type=f32,
                precision=hi)

        ms_col = to_col(ms)
        areas_col = jnp.sum(inter * eye, axis=1, keepdims=True)
        areas_row = jnp.sum(inter * eye, axis=0, keepdims=True)
        union = jnp.maximum(areas_col + areas_row - inter, 1.0)
        iou = inter / union

        gt = (ms_col > ms).astype(f32)
        tie = ((ms_col == ms) & (row_i < col_i)).astype(f32)
        rank_row = jnp.sum(gt + tie, axis=0, keepdims=True)
        p = (row_i.astype(f32) == rank_row).astype(f32)

        tmp = jax.lax.dot_general(
            p, iou, (((1,), (0,)), ((), ())), preferred_element_type=f32,
            precision=hi)
        iou_s = jax.lax.dot_general(
            tmp, p, (((1,), (1,)), ((), ())), preferred_element_type=f32,
            precision=hi)
        iou_scr[...] = iou_s

        s_col = jnp.sum(p * ms, axis=1, keepdims=True)
        valid_col = s_col > SELECT_THR
        total = jnp.sum(valid_col.astype(f32))
        first = jax.lax.broadcasted_iota(jnp.int32, (N, 1), 0) == 0
        valid_col = valid_col | (first & (total == 0.0))

        lane = jax.lax.broadcasted_iota(jnp.int32, (1, N), 1)
        ones = jnp.ones((1, N), f32)

        def body(i, keep):
            row = iou_scr[pl.ds(i, 1), :]
            ki = jnp.sum(keep * (lane == i).astype(f32))
            sup = (row > NMS_THR) & (lane > i)
            return keep * (1.0 - sup.astype(f32) * (ki > 0.0).astype(f32))

        any_sup = jnp.max(jnp.where(row_i < col_i, iou_s, 0.0)) > NMS_THR
        keep = jax.lax.cond(
            any_sup, lambda: jax.lax.fori_loop(0, N, body, ones), lambda: ones)
        p_scr[...] = p
        w_scr[...] = to_col(keep) * valid_col.astype(f32) * s_col

    @pl.when(g > NB)
    def _phase_c():
        j = g - (NB + 1)
        slot = jax.lax.rem(j, 2)
        pb = p_scr[...].astype(jnp.bfloat16)
        w = w_scr[...][:, :, None]

        @pl.when(j >= 2)
        def _():
            for s in range(2):
                pltpu.make_async_copy(
                    obuf_scr.at[s, slot],
                    out_ref.at[:, pl.ds(HB * (j - 2 + s * NB), HB), :],
                    sem.at[s, slot]).wait()

        for s in range(2):
            blk = j + s * NB
            acc = jax.lax.dot_general(
                pb, sig_scr[blk], (((1,), (0,)), ((), ())),
                preferred_element_type=f32)
            obuf_scr[s, slot] = acc * w
            pltpu.make_async_copy(
                obuf_scr.at[s, slot],
                out_ref.at[:, pl.ds(HB * blk, HB), :],
                sem.at[s, slot]).start()

        @pl.when(j == NB - 1)
        def _():
            for s in range(2):
                for sl in range(2):
                    blk = j - 1 + sl + s * NB
                    pltpu.make_async_copy(
                        obuf_scr.at[s, sl],
                        out_ref.at[:, pl.ds(HB * blk, HB), :],
                        sem.at[s, sl]).wait()


def kernel(pred_logits, pred_masks):
    scores = jax.nn.softmax(pred_logits, axis=-1)[:, :-1]
    ms_row = jnp.max(scores, axis=1).reshape(1, N)

    out = pl.pallas_call(
        _tracker_kernel,
        grid=(2 * NB + 1,),
        in_specs=[
            pl.BlockSpec((1, N), lambda g: (0, 0)),
            pl.BlockSpec((N, HB, W),
                         lambda g: (0, jnp.minimum(g, NB - 1), 0)),
            pl.BlockSpec((N, HB, W),
                         lambda g: (0, jnp.minimum(g, NB - 1) + NB, 0)),
        ],
        out_specs=pl.BlockSpec(memory_space=pl.ANY),
        out_shape=jax.ShapeDtypeStruct((N, H, W), jnp.float32),
        scratch_shapes=[
            pltpu.VMEM((2 * NB, N, HB, W), jnp.bfloat16),
            pltpu.VMEM((N, N), jnp.float32),
            pltpu.VMEM((N, N), jnp.float32),
            pltpu.VMEM((N, 1), jnp.float32),
            pltpu.VMEM((N, N), jnp.float32),
            pltpu.VMEM((2, 2, N, HB, W), jnp.float32),
            pltpu.SemaphoreType.DMA((2, 2)),
        ],
    )(ms_row, pred_masks, pred_masks)

    return out
